# speedup vs baseline: 6.2468x; 6.2468x over previous
import functools

import jax
import jax.numpy as jnp
from jax import lax
from jax.experimental import pallas as pl
from jax.experimental.pallas import tpu as pltpu
from jax.experimental.pallas import tpu_sc as plsc

B = 4096
L = 200
D = 128
NC = 2
NS = 16
NW = NC * NS
BPW = B // NW
C1 = 128
C2 = L - C1
NCH = D // 16


def _build():
  mesh = plsc.VectorSubcoreMesh(core_axis_name="c", subcore_axis_name="s")

  @functools.partial(
      pl.kernel,
      out_type=jax.ShapeDtypeStruct((B, D), jnp.float32),
      mesh=mesh,
      scratch_types=[
          pltpu.VMEM((BPW * L,), jnp.int32),
          pltpu.VMEM((C1, D), jnp.float32),
          pltpu.VMEM((C2, D), jnp.float32),
          pltpu.VMEM((BPW, D), jnp.float32),
          pltpu.SemaphoreType.DMA,
      ],
  )
  def k(table_hbm, idx_hbm, out_hbm, idx_v, rows1_v, rows2_v, out_v, sem):
    wid = lax.axis_index("c") * NS + lax.axis_index("s")
    base = wid * BPW
    pltpu.sync_copy(idx_hbm.at[pl.ds(base * L, BPW * L)], idx_v)

    @pl.loop(0, BPW)
    def _bag(b):
      off = pl.multiple_of(b * L, 8)
      pltpu.sync_copy(table_hbm.at[idx_v.at[pl.ds(off, C1)]], rows1_v)
      pltpu.sync_copy(table_hbm.at[idx_v.at[pl.ds(off + C1, C2)]], rows2_v)

      def add1(r, accs):
        return tuple(accs[c] + rows1_v[r, pl.ds(c * 16, 16)]
                     for c in range(NCH))

      def add2(r, accs):
        return tuple(accs[c] + rows2_v[r, pl.ds(c * 16, 16)]
                     for c in range(NCH))

      accs = tuple(rows1_v[0, pl.ds(c * 16, 16)] for c in range(NCH))
      accs = lax.fori_loop(1, C1, add1, accs)
      accs = lax.fori_loop(0, C2, add2, accs)
      scale = jnp.float32(1.0 / L)
      for c in range(NCH):
        out_v[b, pl.ds(c * 16, 16)] = accs[c] * scale

    pltpu.sync_copy(out_v, out_hbm.at[pl.ds(base, BPW)])

  return k


def kernel(sentences, offsets, weight):
  del offsets
  idx_flat = sentences.reshape(-1)
  return _build()(weight, idx_flat)


# speedup vs baseline: 13.4453x; 2.1523x over previous
import functools

import jax
import jax.numpy as jnp
from jax import lax
from jax.experimental import pallas as pl
from jax.experimental.pallas import tpu as pltpu
from jax.experimental.pallas import tpu_sc as plsc

B = 4096
L = 200
D = 128
NC = 2
NS = 16
NW = NC * NS
BPW = B // NW
C1 = 128
C2 = L - C1
NCH = D // 16


def _build():
  mesh = plsc.VectorSubcoreMesh(core_axis_name="c", subcore_axis_name="s")

  @functools.partial(
      pl.kernel,
      out_type=jax.ShapeDtypeStruct((B, D), jnp.float32),
      mesh=mesh,
      scratch_types=[
          pltpu.VMEM((BPW * L,), jnp.int32),
          pltpu.VMEM((2, C1, D), jnp.float32),
          pltpu.VMEM((2, C2, D), jnp.float32),
          pltpu.VMEM((BPW, D), jnp.float32),
          pltpu.SemaphoreType.DMA,
          pltpu.SemaphoreType.DMA,
      ],
  )
  def k(table_hbm, idx_hbm, out_hbm, idx_v, rows1_v, rows2_v, out_v,
        sem0, sem1):
    wid = lax.axis_index("c") * NS + lax.axis_index("s")
    base = wid * BPW
    pltpu.sync_copy(idx_hbm.at[pl.ds(base * L, BPW * L)], idx_v)
    sems = (sem0, sem1)

    def start(bb, buf):
      off = pl.multiple_of(bb * L, 8)
      pltpu.async_copy(table_hbm.at[idx_v.at[pl.ds(off, C1)]],
                       rows1_v.at[buf], sems[buf])
      pltpu.async_copy(table_hbm.at[idx_v.at[pl.ds(off + C1, C2)]],
                       rows2_v.at[buf], sems[buf])

    def wait(bb, buf):
      off = pl.multiple_of(bb * L, 8)
      pltpu.make_async_copy(table_hbm.at[idx_v.at[pl.ds(off, C1)]],
                            rows1_v.at[buf], sems[buf]).wait()
      pltpu.make_async_copy(table_hbm.at[idx_v.at[pl.ds(off + C1, C2)]],
                            rows2_v.at[buf], sems[buf]).wait()

    start(0, 0)

    @pl.loop(0, BPW, step=2)
    def _pair(b):
      for ph in range(2):
        bb = b + ph

        @pl.when(bb + 1 < BPW)
        def _():
          start(bb + 1, 1 - ph)

        wait(bb, ph)
        r1 = rows1_v.at[ph]
        r2 = rows2_v.at[ph]

        def add1(r, accs):
          return tuple(accs[c] + r1[r, pl.ds(c * 16, 16)]
                       for c in range(NCH))

        def add2(r, accs):
          return tuple(accs[c] + r2[r, pl.ds(c * 16, 16)]
                       for c in range(NCH))

        accs = tuple(r1[0, pl.ds(c * 16, 16)] for c in range(NCH))
        accs = lax.fori_loop(1, C1, add1, accs, unroll=4)
        accs = lax.fori_loop(0, C2, add2, accs, unroll=4)
        scale = jnp.float32(1.0 / L)
        for c in range(NCH):
          out_v[bb, pl.ds(c * 16, 16)] = accs[c] * scale

    pltpu.sync_copy(out_v, out_hbm.at[pl.ds(base, BPW)])

  return k


def kernel(sentences, offsets, weight):
  del offsets
  idx_flat = sentences.reshape(-1)
  return _build()(weight, idx_flat)
